# baseline (speedup 1.0000x reference)
import jax
import jax.numpy as jnp
from jax import lax
from jax.experimental import pallas as pl
from jax.experimental.pallas import tpu as pltpu
from jax.experimental.pallas import tpu_sc as plsc

VX, VY = 0.16, 0.16
PC_RANGE = [0.0, -39.68, -3.0, 69.12, 39.68, 1.0]
X_OFFSET = VX / 2 + PC_RANGE[0]
Y_OFFSET = VY / 2 + PC_RANGE[1]
X_L = 432
Y_L = 496
MAXP = 32
P = 40000
P_PAD = 40960
B = 2
OUT_DIM = 64
EPS = 1e-3
ZCONST = (PC_RANGE[5] + PC_RANGE[2]) / 2.0
YX = Y_L * X_L
PLANE_B = OUT_DIM * YX
NEG = -1e30

NP_BLK = 1024
N_BLK = P_PAD // NP_BLK


def _p1_body(pil_ref, coors_ref, npp_ref, cx_ref, cy_ref, cb_ref,
             rhs_ref, s8_ref, wct_ref, t_ref, xmax_ref, idx_ref):
    pil = pil_ref[...]
    scores = jnp.dot(pil, rhs_ref[...],
                     preferred_element_type=jnp.float32)
    npp = npp_ref[...]
    jlane = lax.broadcasted_iota(jnp.int32, (1, 2048), 1) // OUT_DIM
    masked = jnp.where(jlane < npp, scores, NEG)
    m = masked
    w = 1024
    while w >= OUT_DIM:
        m = jnp.maximum(m[:, :w], m[:, w:2 * w])
        w //= 2
    sums = jnp.dot(pil, s8_ref[...],
                   preferred_element_type=jnp.float32)
    nppf = npp.astype(jnp.float32)
    coors = coors_ref[...]
    cf = coors.astype(jnp.float32)
    cxf = cf[:, 0:1] * VX + X_OFFSET
    cyf = cf[:, 1:2] * VY + Y_OFFSET
    mx = sums[:, 0:1] / nppf
    my = sums[:, 1:2] / nppf
    mz = sums[:, 2:3] / nppf
    ones = jnp.ones_like(cxf)
    zer = jnp.zeros_like(cxf)
    cp = jnp.concatenate([cxf, cyf, mx, my, mz, ones, zer, zer], axis=1)
    bias = jnp.dot(cp, wct_ref[...],
                   preferred_element_type=jnp.float32)
    cand0 = jnp.where(npp < MAXP, 0.0, NEG)
    m3 = jnp.maximum(m + bias, cand0)
    out = jnp.maximum(m3 + t_ref[...], 0.0)
    xmax_ref[...] = jnp.concatenate(
        [out, jnp.zeros((NP_BLK, 64), jnp.float32)], axis=1)
    idx_ref[...] = (cb_ref[...] * YX + cy_ref[...] * X_L + cx_ref[...])


def _phase1(pillars128, coors, npp2, cx2d, cy2d, cb2d, rhs, s8, wct, tvec):
    return pl.pallas_call(
        _p1_body,
        grid=(N_BLK,),
        in_specs=[
            pl.BlockSpec((NP_BLK, 128), lambda i: (i, 0)),
            pl.BlockSpec((NP_BLK, 4), lambda i: (i, 0)),
            pl.BlockSpec((NP_BLK, 1), lambda i: (i, 0)),
            pl.BlockSpec((8, 128), lambda i: (i, 0)),
            pl.BlockSpec((8, 128), lambda i: (i, 0)),
            pl.BlockSpec((8, 128), lambda i: (i, 0)),
            pl.BlockSpec((128, 2048), lambda i: (0, 0)),
            pl.BlockSpec((128, 8), lambda i: (0, 0)),
            pl.BlockSpec((8, 64), lambda i: (0, 0)),
            pl.BlockSpec((1, 64), lambda i: (0, 0)),
        ],
        out_specs=[
            pl.BlockSpec((NP_BLK, 128), lambda i: (i, 0)),
            pl.BlockSpec((8, 128), lambda i: (i, 0)),
        ],
        out_shape=[
            jax.ShapeDtypeStruct((P_PAD, 128), jnp.float32),
            jax.ShapeDtypeStruct((P_PAD // 128, 128), jnp.int32),
        ],
    )(pillars128, coors, npp2, cx2d, cy2d, cb2d, rhs, s8, wct, tvec)


NC = 2
NS = 16
NW = NC * NS
CELLS_T = (B * YX) // NW
CCH = 1024
NCCH = 14
GB = 128
BUFW = CCH + 8
DUMP = CCH + 1
WMAP = 13824


def _p2_body(xmax_hbm, idx_hbm, out_hbm, win, ibuf, cpil, ccell, cp2,
             gbuf, buf, sem):
    t = lax.axis_index("s") * NC + lax.axis_index("c")
    cellbase = t * CELLS_T

    neg1 = jnp.full((16,), -1, jnp.int32)

    def wz(i, _):
        win[pl.ds(i * 16, 16)] = neg1
        return 0
    lax.fori_loop(0, WMAP // 16, wz, 0)
    zf = jnp.zeros((16,), jnp.float32)

    def bz(i, _):
        buf[pl.ds(i * 16, 16)] = zf
        return 0
    lax.fori_loop(0, (OUT_DIM * BUFW) // 16, bz, 0)

    iota16 = lax.broadcasted_iota(jnp.int32, (16,), 0)

    for kb in range(P_PAD // 2048):
        pltpu.sync_copy(idx_hbm.at[pl.ds(kb * 2048, 2048)], ibuf)
        base0 = kb * 2048

        def claim(v, _):
            cellv = ibuf[pl.ds(v * 16, 16)]
            localv = cellv - cellbase
            pids = base0 + v * 16 + iota16
            m = (localv >= 0) & (localv < CELLS_T) & (pids < P)
            plsc.store_scatter(win, [localv], pids, mask=m)
            return 0
        lax.fori_loop(0, 128, claim, 0)

    b64 = (t // NS) * OUT_DIM
    tbase = (t % NS) * CELLS_T

    def do_chunk(ci, csz):
        def comp(q, cnt):
            wv = win[pl.ds(ci * CCH + q * 16, 16)]
            m = wv >= 0
            cells = q * 16 + iota16
            cs = jnp.cumsum(m.astype(jnp.int32))
            pos = cnt + cs - 1
            plsc.store_scatter(cpil, [pos], wv, mask=m)
            plsc.store_scatter(ccell, [pos], cells, mask=m)
            return cnt + jnp.max(cs)
        cnt = lax.fori_loop(0, csz // 16, comp, jnp.int32(0))
        for u in range(GB // 16):
            cpil[pl.ds(cnt + u * 16, 16)] = jnp.zeros((16,), jnp.int32)
            ccell[pl.ds(cnt + u * 16, 16)] = jnp.full((16,), DUMP, jnp.int32)
        nb = (cnt + GB - 1) // GB

        def gather_place(g, _):
            def cpy(i, _3):
                cp2[pl.ds(i * 16, 16)] = cpil[pl.ds(g * GB + i * 16, 16)]
                return 0
            lax.fori_loop(0, GB // 16, cpy, 0)
            pltpu.async_copy(xmax_hbm.at[cp2], gbuf, sem).wait()

            def place(r, _2):
                cell = ccell[pl.ds(g * GB + r, 16)][0]
                for k in range(4):
                    vals = gbuf[r, pl.ds(k * 16, 16)]
                    tix = (iota16 + k * 16) * BUFW + cell
                    plsc.store_scatter(buf, [tix], vals)
                return 0
            lax.fori_loop(0, GB, place, 0)
            return 0
        lax.fori_loop(0, nb, gather_place, 0)

        dst0 = tbase + ci * CCH
        sdescs = []
        for c in range(OUT_DIM):
            sdescs.append(pltpu.async_copy(
                buf.at[pl.ds(c * BUFW, csz)],
                out_hbm.at[pl.ds((b64 + c) * YX + dst0, csz)], sem))
        for d in sdescs:
            d.wait()

        def rz(g, _):
            def rzp(r, _2):
                cell = ccell[pl.ds(g * GB + r, 16)][0]
                for k in range(4):
                    tix = (iota16 + k * 16) * BUFW + cell
                    plsc.store_scatter(buf, [tix],
                                       jnp.zeros((16,), jnp.float32))
                return 0
            lax.fori_loop(0, GB, rzp, 0)
            return 0
        lax.fori_loop(0, nb, rz, 0)

    def chunk_body(ci, _):
        do_chunk(ci, CCH)
        return 0
    lax.fori_loop(0, NCCH - 1, chunk_body, 0)
    do_chunk(NCCH - 1, CELLS_T - (NCCH - 1) * CCH)


def _phase2(xmax, idx2d):
    mesh = plsc.VectorSubcoreMesh(core_axis_name="c", subcore_axis_name="s",
                                  num_cores=NC, num_subcores=NS)
    return pl.kernel(
        _p2_body,
        out_type=jax.ShapeDtypeStruct((B * OUT_DIM * YX,), jnp.float32),
        mesh=mesh,
        compiler_params=pltpu.CompilerParams(needs_layout_passes=False),
        scratch_types=[
            pltpu.VMEM((WMAP,), jnp.int32),
            pltpu.VMEM((2048,), jnp.int32),
            pltpu.VMEM((CCH + 2 * GB,), jnp.int32),
            pltpu.VMEM((CCH + 2 * GB,), jnp.int32),
            pltpu.VMEM((GB,), jnp.int32),
            pltpu.VMEM((GB, 128), jnp.float32),
            pltpu.VMEM((OUT_DIM * BUFW,), jnp.float32),
            pltpu.SemaphoreType.DMA,
        ],
    )(xmax, idx2d)


def kernel(pillars, coors_batch, npoints_per_pillar, W, bn_gamma, bn_beta,
           bn_mean, bn_var):
    f32 = jnp.float32
    s = bn_gamma / jnp.sqrt(bn_var + EPS)
    t = bn_beta - bn_mean * s
    wx = (W[:, 0] + W[:, 4] + W[:, 7]) * s
    wy = (W[:, 1] + W[:, 5] + W[:, 8]) * s
    wz = (W[:, 2] + W[:, 6]) * s
    wr = W[:, 3] * s
    w4s = jnp.stack([wx, wy, wz, wr], axis=0)
    rhs = jnp.kron(jnp.eye(32, dtype=f32), w4s)
    s8 = jnp.tile(jnp.eye(4, dtype=f32), (32, 1))
    s8 = jnp.concatenate([s8, jnp.zeros((128, 4), f32)], axis=1)
    wct = jnp.stack([
        -(W[:, 0] + W[:, 7]) * s,
        -(W[:, 1] + W[:, 8]) * s,
        -W[:, 4] * s,
        -W[:, 5] * s,
        -W[:, 6] * s,
        ZCONST * W[:, 9] * s,
        jnp.zeros_like(s),
        jnp.zeros_like(s),
    ], axis=0)
    tvec = t.reshape(1, OUT_DIM)

    npad = P_PAD - P
    pillars128 = jnp.pad(pillars.reshape(P, 128), ((0, npad), (0, 0)))
    coorsp = jnp.pad(coors_batch, ((0, npad), (0, 0)))
    npp2 = jnp.pad(npoints_per_pillar.reshape(P, 1), ((0, npad), (0, 0)),
                   constant_values=1)
    cx2d = coorsp[:, 0].reshape(P_PAD // 128, 128)
    cy2d = coorsp[:, 1].reshape(P_PAD // 128, 128)
    cb2d = coorsp[:, 3].reshape(P_PAD // 128, 128)

    xmax, idx2d = _phase1(pillars128, coorsp, npp2, cx2d, cy2d, cb2d,
                          rhs, s8, wct, tvec)
    out_flat = _phase2(xmax, idx2d.reshape(P_PAD))
    return jnp.maximum(out_flat.reshape(B, OUT_DIM, Y_L, X_L), 0.0)


# speedup vs baseline: 1.0620x; 1.0620x over previous
import jax
import jax.numpy as jnp
from jax import lax
from jax.experimental import pallas as pl
from jax.experimental.pallas import tpu as pltpu
from jax.experimental.pallas import tpu_sc as plsc

VX, VY = 0.16, 0.16
PC_RANGE = [0.0, -39.68, -3.0, 69.12, 39.68, 1.0]
X_OFFSET = VX / 2 + PC_RANGE[0]
Y_OFFSET = VY / 2 + PC_RANGE[1]
X_L = 432
Y_L = 496
MAXP = 32
P = 40000
P_PAD = 40960
B = 2
OUT_DIM = 64
EPS = 1e-3
ZCONST = (PC_RANGE[5] + PC_RANGE[2]) / 2.0
YX = Y_L * X_L
PLANE_B = OUT_DIM * YX
NEG = -1e30

NP_BLK = 1024
N_BLK = P_PAD // NP_BLK


def _p1_body(pil_ref, coors_ref, npp_ref, cx_ref, cy_ref, cb_ref,
             rhs_ref, s8_ref, wct_ref, t_ref, xmax_ref, idx_ref):
    pil = pil_ref[...]
    scores = jnp.dot(pil, rhs_ref[...],
                     preferred_element_type=jnp.float32)
    npp = npp_ref[...]
    jlane = lax.broadcasted_iota(jnp.int32, (1, 2048), 1) // OUT_DIM
    masked = jnp.where(jlane < npp, scores, NEG)
    m = masked
    w = 1024
    while w >= OUT_DIM:
        m = jnp.maximum(m[:, :w], m[:, w:2 * w])
        w //= 2
    sums = jnp.dot(pil, s8_ref[...],
                   preferred_element_type=jnp.float32)
    nppf = npp.astype(jnp.float32)
    coors = coors_ref[...]
    cf = coors.astype(jnp.float32)
    cxf = cf[:, 0:1] * VX + X_OFFSET
    cyf = cf[:, 1:2] * VY + Y_OFFSET
    mx = sums[:, 0:1] / nppf
    my = sums[:, 1:2] / nppf
    mz = sums[:, 2:3] / nppf
    ones = jnp.ones_like(cxf)
    zer = jnp.zeros_like(cxf)
    cp = jnp.concatenate([cxf, cyf, mx, my, mz, ones, zer, zer], axis=1)
    bias = jnp.dot(cp, wct_ref[...],
                   preferred_element_type=jnp.float32)
    cand0 = jnp.where(npp < MAXP, 0.0, NEG)
    m3 = jnp.maximum(m + bias, cand0)
    out = jnp.maximum(m3 + t_ref[...], 0.0)
    xmax_ref[...] = jnp.concatenate(
        [out, jnp.zeros((NP_BLK, 64), jnp.float32)], axis=1)
    idx_ref[...] = (cb_ref[...] * YX + cy_ref[...] * X_L + cx_ref[...])


def _phase1(pillars128, coors, npp2, cx2d, cy2d, cb2d, rhs, s8, wct, tvec):
    return pl.pallas_call(
        _p1_body,
        grid=(N_BLK,),
        in_specs=[
            pl.BlockSpec((NP_BLK, 128), lambda i: (i, 0)),
            pl.BlockSpec((NP_BLK, 4), lambda i: (i, 0)),
            pl.BlockSpec((NP_BLK, 1), lambda i: (i, 0)),
            pl.BlockSpec((8, 128), lambda i: (i, 0)),
            pl.BlockSpec((8, 128), lambda i: (i, 0)),
            pl.BlockSpec((8, 128), lambda i: (i, 0)),
            pl.BlockSpec((128, 2048), lambda i: (0, 0)),
            pl.BlockSpec((128, 8), lambda i: (0, 0)),
            pl.BlockSpec((8, 64), lambda i: (0, 0)),
            pl.BlockSpec((1, 64), lambda i: (0, 0)),
        ],
        out_specs=[
            pl.BlockSpec((NP_BLK, 128), lambda i: (i, 0)),
            pl.BlockSpec((8, 128), lambda i: (i, 0)),
        ],
        out_shape=[
            jax.ShapeDtypeStruct((P_PAD, 128), jnp.float32),
            jax.ShapeDtypeStruct((P_PAD // 128, 128), jnp.int32),
        ],
    )(pillars128, coors, npp2, cx2d, cy2d, cb2d, rhs, s8, wct, tvec)


NC = 2
NS = 16
NW = NC * NS
SLAB = 13 * 1024
SLAB_LAST = YX - 15 * SLAB
CCH = 1024
TAILC = SLAB_LAST - 14 * CCH
GB = 128
BUFW = CCH + 16
DUMP = CCH + 1
WMAP = SLAB_LAST


def _p2_body(xmax_hbm, idx_hbm, out_hbm, win, ibuf, cpil, ccell, cp2,
             gbuf, buf, sem):
    t = lax.axis_index("s") * NC + lax.axis_index("c")
    k = t % NS
    last = k == NS - 1
    slabsize = jnp.where(last, SLAB_LAST, SLAB)
    tbase = k * SLAB
    cellbase = (t // NS) * YX + tbase

    neg1 = jnp.full((16,), -1, jnp.int32)

    def wz(i, _):
        win[pl.ds(i * 16, 16)] = neg1
        return 0
    lax.fori_loop(0, WMAP // 16, wz, 0)
    zf = jnp.zeros((16,), jnp.float32)
    for cch in range(OUT_DIM):
        def bz(i, _):
            buf[cch, pl.ds(i * 16, 16)] = zf
            return 0
        lax.fori_loop(0, BUFW // 16, bz, 0)

    iota16 = lax.broadcasted_iota(jnp.int32, (16,), 0)

    for kb in range(P_PAD // 2048):
        pltpu.sync_copy(idx_hbm.at[pl.ds(kb * 2048, 2048)], ibuf)
        base0 = kb * 2048

        def claim(v, _):
            cellv = ibuf[pl.ds(v * 16, 16)]
            localv = cellv - cellbase
            pids = base0 + v * 16 + iota16
            m = (localv >= 0) & (localv < slabsize) & (pids < P)
            plsc.store_scatter(win, [localv], pids, mask=m)
            return 0
        lax.fori_loop(0, 128, claim, 0)

    b64 = (t // NS) * OUT_DIM

    def do_chunk(ci, csz):
        def comp(q, cnt):
            wv = win[pl.ds(ci * CCH + q * 16, 16)]
            m = wv >= 0
            cells = q * 16 + iota16
            cs = jnp.cumsum(m.astype(jnp.int32))
            pos = cnt + cs - 1
            plsc.store_scatter(cpil, [pos], wv, mask=m)
            plsc.store_scatter(ccell, [pos], cells, mask=m)
            return cnt + jnp.max(cs)
        cnt = lax.fori_loop(0, csz // 16, comp, jnp.int32(0))
        for u in range(GB // 16):
            cpil[pl.ds(cnt + u * 16, 16)] = jnp.zeros((16,), jnp.int32)
            ccell[pl.ds(cnt + u * 16, 16)] = jnp.full((16,), DUMP, jnp.int32)
        nb = (cnt + GB - 1) // GB

        def gather_place(g, _):
            def cpy(i, _3):
                cp2[pl.ds(i * 16, 16)] = cpil[pl.ds(g * GB + i * 16, 16)]
                return 0
            lax.fori_loop(0, GB // 16, cpy, 0)
            pltpu.async_copy(xmax_hbm.at[cp2], gbuf, sem).wait()

            def place(r, _2):
                cellv = ccell[pl.ds(g * GB + r, 16)][0] + jnp.zeros(
                    (16,), jnp.int32)
                for k in range(4):
                    vals = gbuf[r, pl.ds(k * 16, 16)]
                    plsc.store_scatter(buf, [iota16 + k * 16, cellv], vals)
                return 0
            lax.fori_loop(0, GB, place, 0)
            return 0
        lax.fori_loop(0, nb, gather_place, 0)

        dst0 = tbase + ci * CCH
        pltpu.async_copy(
            buf.at[:, pl.ds(0, csz)],
            out_hbm.at[pl.ds(b64, OUT_DIM), pl.ds(dst0, csz)],
            sem).wait()

        def rz(g, _):
            def rzp(r, _2):
                cellv = ccell[pl.ds(g * GB + r, 16)][0] + jnp.zeros(
                    (16,), jnp.int32)
                for k in range(4):
                    plsc.store_scatter(buf, [iota16 + k * 16, cellv],
                                       jnp.zeros((16,), jnp.float32))
                return 0
            lax.fori_loop(0, GB, rzp, 0)
            return 0
        lax.fori_loop(0, nb, rz, 0)

    def chunk_body(ci, _):
        do_chunk(ci, CCH)
        return 0
    nfull = jnp.where(last, 14, 13)
    lax.fori_loop(0, nfull, chunk_body, 0)

    @pl.when(last)
    def _tail():
        do_chunk(14, TAILC)


def _phase2(xmax, idx2d):
    mesh = plsc.VectorSubcoreMesh(core_axis_name="c", subcore_axis_name="s",
                                  num_cores=NC, num_subcores=NS)
    return pl.kernel(
        _p2_body,
        out_type=jax.ShapeDtypeStruct((B * OUT_DIM, YX), jnp.float32),
        mesh=mesh,
        compiler_params=pltpu.CompilerParams(needs_layout_passes=False),
        scratch_types=[
            pltpu.VMEM((WMAP,), jnp.int32),
            pltpu.VMEM((2048,), jnp.int32),
            pltpu.VMEM((CCH + 2 * GB,), jnp.int32),
            pltpu.VMEM((CCH + 2 * GB,), jnp.int32),
            pltpu.VMEM((GB,), jnp.int32),
            pltpu.VMEM((GB, 128), jnp.float32),
            pltpu.VMEM((OUT_DIM, BUFW), jnp.float32),
            pltpu.SemaphoreType.DMA,
        ],
    )(xmax, idx2d)


def kernel(pillars, coors_batch, npoints_per_pillar, W, bn_gamma, bn_beta,
           bn_mean, bn_var):
    f32 = jnp.float32
    s = bn_gamma / jnp.sqrt(bn_var + EPS)
    t = bn_beta - bn_mean * s
    wx = (W[:, 0] + W[:, 4] + W[:, 7]) * s
    wy = (W[:, 1] + W[:, 5] + W[:, 8]) * s
    wz = (W[:, 2] + W[:, 6]) * s
    wr = W[:, 3] * s
    w4s = jnp.stack([wx, wy, wz, wr], axis=0)
    rhs = jnp.kron(jnp.eye(32, dtype=f32), w4s)
    s8 = jnp.tile(jnp.eye(4, dtype=f32), (32, 1))
    s8 = jnp.concatenate([s8, jnp.zeros((128, 4), f32)], axis=1)
    wct = jnp.stack([
        -(W[:, 0] + W[:, 7]) * s,
        -(W[:, 1] + W[:, 8]) * s,
        -W[:, 4] * s,
        -W[:, 5] * s,
        -W[:, 6] * s,
        ZCONST * W[:, 9] * s,
        jnp.zeros_like(s),
        jnp.zeros_like(s),
    ], axis=0)
    tvec = t.reshape(1, OUT_DIM)

    npad = P_PAD - P
    pillars128 = jnp.pad(pillars.reshape(P, 128), ((0, npad), (0, 0)))
    coorsp = jnp.pad(coors_batch, ((0, npad), (0, 0)))
    npp2 = jnp.pad(npoints_per_pillar.reshape(P, 1), ((0, npad), (0, 0)),
                   constant_values=1)
    cx2d = coorsp[:, 0].reshape(P_PAD // 128, 128)
    cy2d = coorsp[:, 1].reshape(P_PAD // 128, 128)
    cb2d = coorsp[:, 3].reshape(P_PAD // 128, 128)

    xmax, idx2d = _phase1(pillars128, coorsp, npp2, cx2d, cy2d, cb2d,
                          rhs, s8, wct, tvec)
    out_flat = _phase2(xmax, idx2d.reshape(P_PAD))
    return out_flat.reshape(B, OUT_DIM, Y_L, X_L)


# speedup vs baseline: 1.4736x; 1.3876x over previous
import jax
import jax.numpy as jnp
from jax import lax
from jax.experimental import pallas as pl
from jax.experimental.pallas import tpu as pltpu
from jax.experimental.pallas import tpu_sc as plsc

VX, VY = 0.16, 0.16
PC_RANGE = [0.0, -39.68, -3.0, 69.12, 39.68, 1.0]
X_OFFSET = VX / 2 + PC_RANGE[0]
Y_OFFSET = VY / 2 + PC_RANGE[1]
X_L = 432
Y_L = 496
MAXP = 32
P = 40000
P_PAD = 40960
B = 2
OUT_DIM = 64
EPS = 1e-3
ZCONST = (PC_RANGE[5] + PC_RANGE[2]) / 2.0
YX = Y_L * X_L
PLANE_B = OUT_DIM * YX
NEG = -1e30

NP_BLK = 1024
N_BLK = P_PAD // NP_BLK


def _p1_body(pil_ref, coors_ref, npp_ref, cx_ref, cy_ref, cb_ref,
             rhs_ref, s8_ref, wct_ref, t_ref, xmax_ref, idx_ref):
    pil = pil_ref[...]
    scores = jnp.dot(pil, rhs_ref[...],
                     preferred_element_type=jnp.float32)
    npp = npp_ref[...]
    jlane = lax.broadcasted_iota(jnp.int32, (1, 2048), 1) // OUT_DIM
    masked = jnp.where(jlane < npp, scores, NEG)
    m = masked
    w = 1024
    while w >= OUT_DIM:
        m = jnp.maximum(m[:, :w], m[:, w:2 * w])
        w //= 2
    sums = jnp.dot(pil, s8_ref[...],
                   preferred_element_type=jnp.float32)
    nppf = npp.astype(jnp.float32)
    coors = coors_ref[...]
    cf = coors.astype(jnp.float32)
    cxf = cf[:, 0:1] * VX + X_OFFSET
    cyf = cf[:, 1:2] * VY + Y_OFFSET
    mx = sums[:, 0:1] / nppf
    my = sums[:, 1:2] / nppf
    mz = sums[:, 2:3] / nppf
    ones = jnp.ones_like(cxf)
    zer = jnp.zeros_like(cxf)
    cp = jnp.concatenate([cxf, cyf, mx, my, mz, ones, zer, zer], axis=1)
    bias = jnp.dot(cp, wct_ref[...],
                   preferred_element_type=jnp.float32)
    cand0 = jnp.where(npp < MAXP, 0.0, NEG)
    m3 = jnp.maximum(m + bias, cand0)
    out = jnp.maximum(m3 + t_ref[...], 0.0)
    xmax_ref[...] = jnp.concatenate(
        [out, jnp.zeros((NP_BLK, 64), jnp.float32)], axis=1)
    idx_ref[...] = (cb_ref[...] * YX + cy_ref[...] * X_L + cx_ref[...])


def _phase1(pillars128, coors, npp2, cx2d, cy2d, cb2d, rhs, s8, wct, tvec):
    return pl.pallas_call(
        _p1_body,
        grid=(N_BLK,),
        in_specs=[
            pl.BlockSpec((NP_BLK, 128), lambda i: (i, 0)),
            pl.BlockSpec((NP_BLK, 4), lambda i: (i, 0)),
            pl.BlockSpec((NP_BLK, 1), lambda i: (i, 0)),
            pl.BlockSpec((8, 128), lambda i: (i, 0)),
            pl.BlockSpec((8, 128), lambda i: (i, 0)),
            pl.BlockSpec((8, 128), lambda i: (i, 0)),
            pl.BlockSpec((128, 2048), lambda i: (0, 0)),
            pl.BlockSpec((128, 8), lambda i: (0, 0)),
            pl.BlockSpec((8, 64), lambda i: (0, 0)),
            pl.BlockSpec((1, 64), lambda i: (0, 0)),
        ],
        out_specs=[
            pl.BlockSpec((NP_BLK, 128), lambda i: (i, 0)),
            pl.BlockSpec((8, 128), lambda i: (i, 0)),
        ],
        out_shape=[
            jax.ShapeDtypeStruct((P_PAD, 128), jnp.float32),
            jax.ShapeDtypeStruct((P_PAD // 128, 128), jnp.int32),
        ],
    )(pillars128, coors, npp2, cx2d, cy2d, cb2d, rhs, s8, wct, tvec)


NC = 2
NS = 16
NW = NC * NS
SLAB = 13 * 1024
SLAB_LAST = YX - 15 * SLAB
CCH = 1024
TAILC = SLAB_LAST - 14 * CCH
GB = 128
BUFW = CCH + 16
DUMP = CCH + 1
WMAP = SLAB_LAST


def _p2_body(xmax_hbm, idx_hbm, out_hbm, win, ibuf, cpil, ccell, cp2,
             gbuf, buf, sem):
    t = lax.axis_index("s") * NC + lax.axis_index("c")
    k = t % NS
    last = k == NS - 1
    slabsize = jnp.where(last, SLAB_LAST, SLAB)
    tbase = k * SLAB
    cellbase = (t // NS) * YX + tbase

    neg1 = jnp.full((16,), -1, jnp.int32)

    def wz(i, _):
        win[pl.ds(i * 16, 16)] = neg1
        return 0
    lax.fori_loop(0, WMAP // 16, wz, 0)
    zf = jnp.zeros((16,), jnp.float32)
    for cch in range(OUT_DIM):
        def bz(i, _):
            buf[cch, pl.ds(i * 16, 16)] = zf
            return 0
        lax.fori_loop(0, BUFW // 16, bz, 0)

    iota16 = lax.broadcasted_iota(jnp.int32, (16,), 0)

    for kb in range(P_PAD // 2048):
        pltpu.sync_copy(idx_hbm.at[pl.ds(kb * 2048, 2048)], ibuf)
        base0 = kb * 2048

        def claim(v, _):
            cellv = ibuf[pl.ds(v * 16, 16)]
            localv = cellv - cellbase
            pids = base0 + v * 16 + iota16
            m = (localv >= 0) & (localv < slabsize) & (pids < P)
            plsc.store_scatter(win, [localv], pids, mask=m)
            return 0
        lax.fori_loop(0, 128, claim, 0)

    b64 = (t // NS) * OUT_DIM

    def do_chunk(ci, csz):
        def comp(q, cnt):
            wv = win[pl.ds(ci * CCH + q * 16, 16)]
            m = wv >= 0
            cells = q * 16 + iota16
            cs = jnp.cumsum(m.astype(jnp.int32))
            pos = cnt + cs - 1
            plsc.store_scatter(cpil, [pos], wv, mask=m)
            plsc.store_scatter(ccell, [pos], cells, mask=m)
            return cnt + jnp.max(cs)
        cnt = lax.fori_loop(0, csz // 16, comp, jnp.int32(0))
        for u in range(GB // 16):
            cpil[pl.ds(cnt + u * 16, 16)] = jnp.zeros((16,), jnp.int32)
            ccell[pl.ds(cnt + u * 16, 16)] = jnp.full((16,), DUMP, jnp.int32)
        nb = (cnt + GB - 1) // GB

        def gather_place(g, _):
            def cpy(i, _3):
                cp2[pl.ds(i * 16, 16)] = cpil[pl.ds(g * GB + i * 16, 16)]
                return 0
            lax.fori_loop(0, GB // 16, cpy, 0)
            pltpu.async_copy(xmax_hbm.at[cp2], gbuf, sem).wait()

            def place(r, _2):
                cellv = ccell[pl.ds(g * GB + r, 16)][0] + jnp.zeros(
                    (16,), jnp.int32)
                for k in range(4):
                    vals = gbuf[r, pl.ds(k * 16, 16)]
                    plsc.store_scatter(buf, [iota16 + k * 16, cellv], vals)
                return 0
            lax.fori_loop(0, GB, place, 0)
            return 0
        lax.fori_loop(0, nb, gather_place, 0)

        dst0 = tbase + ci * CCH
        pltpu.async_copy(
            buf.at[:, pl.ds(0, csz)],
            out_hbm.at[pl.ds(b64, OUT_DIM), pl.ds(dst0, csz)],
            sem).wait()

        def rz(g, _):
            def rzp(r, _2):
                cellv = ccell[pl.ds(g * GB + r, 16)][0] + jnp.zeros(
                    (16,), jnp.int32)
                for k in range(4):
                    plsc.store_scatter(buf, [iota16 + k * 16, cellv],
                                       jnp.zeros((16,), jnp.float32))
                return 0
            lax.fori_loop(0, GB, rzp, 0)
            return 0
        lax.fori_loop(0, nb, rz, 0)

    def chunk_body(ci, _):
        do_chunk(ci, CCH)
        return 0
    nfull = jnp.where(last, 14, 13)
    lax.fori_loop(0, nfull, chunk_body, 0)

    @pl.when(last)
    def _tail():
        do_chunk(14, TAILC)


def _phase2(xmax, idx2d):
    mesh = plsc.VectorSubcoreMesh(core_axis_name="c", subcore_axis_name="s",
                                  num_cores=NC, num_subcores=NS)
    return pl.kernel(
        _p2_body,
        out_type=jax.ShapeDtypeStruct((B * OUT_DIM, YX), jnp.float32),
        mesh=mesh,
        compiler_params=pltpu.CompilerParams(needs_layout_passes=False),
        scratch_types=[
            pltpu.VMEM((WMAP,), jnp.int32),
            pltpu.VMEM((2048,), jnp.int32),
            pltpu.VMEM((CCH + 2 * GB,), jnp.int32),
            pltpu.VMEM((CCH + 2 * GB,), jnp.int32),
            pltpu.VMEM((GB,), jnp.int32),
            pltpu.VMEM((GB, 128), jnp.float32),
            pltpu.VMEM((OUT_DIM, BUFW), jnp.float32),
            pltpu.SemaphoreType.DMA,
        ],
    )(xmax, idx2d)


def _r_body(canvas_ref, out_ref):
    v = canvas_ref[...]
    out_ref[...] = v.reshape(1, 8, Y_L, X_L)


def _relayout(out2):
    return pl.pallas_call(
        _r_body,
        grid=(B, OUT_DIM // 8),
        in_specs=[pl.BlockSpec((8, YX), lambda b, c: (b * 8 + c, 0))],
        out_specs=pl.BlockSpec((1, 8, Y_L, X_L), lambda b, c: (b, c, 0, 0)),
        out_shape=jax.ShapeDtypeStruct((B, OUT_DIM, Y_L, X_L), jnp.float32),
    )(out2)


def kernel(pillars, coors_batch, npoints_per_pillar, W, bn_gamma, bn_beta,
           bn_mean, bn_var):
    f32 = jnp.float32
    s = bn_gamma / jnp.sqrt(bn_var + EPS)
    t = bn_beta - bn_mean * s
    wx = (W[:, 0] + W[:, 4] + W[:, 7]) * s
    wy = (W[:, 1] + W[:, 5] + W[:, 8]) * s
    wz = (W[:, 2] + W[:, 6]) * s
    wr = W[:, 3] * s
    w4s = jnp.stack([wx, wy, wz, wr], axis=0)
    rhs = jnp.kron(jnp.eye(32, dtype=f32), w4s)
    s8 = jnp.tile(jnp.eye(4, dtype=f32), (32, 1))
    s8 = jnp.concatenate([s8, jnp.zeros((128, 4), f32)], axis=1)
    wct = jnp.stack([
        -(W[:, 0] + W[:, 7]) * s,
        -(W[:, 1] + W[:, 8]) * s,
        -W[:, 4] * s,
        -W[:, 5] * s,
        -W[:, 6] * s,
        ZCONST * W[:, 9] * s,
        jnp.zeros_like(s),
        jnp.zeros_like(s),
    ], axis=0)
    tvec = t.reshape(1, OUT_DIM)

    npad = P_PAD - P
    pillars128 = jnp.pad(pillars.reshape(P, 128), ((0, npad), (0, 0)))
    coorsp = jnp.pad(coors_batch, ((0, npad), (0, 0)))
    npp2 = jnp.pad(npoints_per_pillar.reshape(P, 1), ((0, npad), (0, 0)),
                   constant_values=1)
    cx2d = coorsp[:, 0].reshape(P_PAD // 128, 128)
    cy2d = coorsp[:, 1].reshape(P_PAD // 128, 128)
    cb2d = coorsp[:, 3].reshape(P_PAD // 128, 128)

    xmax, idx2d = _phase1(pillars128, coorsp, npp2, cx2d, cy2d, cb2d,
                          rhs, s8, wct, tvec)
    out2 = _phase2(xmax, idx2d.reshape(P_PAD))
    return _relayout(out2)
